# Initial kernel scaffold; baseline (speedup 1.0000x reference)
import jax
import jax.numpy as jnp
from jax.experimental import pallas as pl


def kernel(x, edge_index, edge_features, global_features, params):
    raise NotImplementedError("write your pallas kernel here")


# speedup vs baseline: 17.5079x; 17.5079x over previous
import functools

import jax
import jax.numpy as jnp
import numpy as np
from jax import lax
from jax.experimental import pallas as pl
from jax.experimental.pallas import tpu as pltpu
from jax.experimental.pallas import tpu_sc as plsc

N = 50000
E = 800000
D = 64
H = 4
C = 16
NUM_CONVS = 3

NP = 51200
EP = 819200
NG = EP // 128
NGC = 416
CROWS = EP + 1024

_f32 = jnp.float32
_i32 = jnp.int32


_BLK = 512


def _full(shape):
    return pl.BlockSpec(shape, lambda *_: tuple(0 for _ in shape))


def _enc_body(x, gf, wn1, bn1, wn2, bn2, wg1, bg1, wg2, bg2, out):
    h = jnp.maximum(jnp.dot(x[...], wn1[...], preferred_element_type=_f32)
                    + bn1[...], 0.0)
    h = jnp.maximum(jnp.dot(h, wn2[...], preferred_element_type=_f32)
                    + bn2[...], 0.0)
    g = jnp.maximum(jnp.dot(gf[...], wg1[...], preferred_element_type=_f32)
                    + bg1[...], 0.0)
    g = jnp.dot(g, wg2[...], preferred_element_type=_f32) + bg2[...]
    out[...] = h + g


def _encoder(xp, gf, p):
    grid = NP // _BLK
    return pl.pallas_call(
        _enc_body,
        grid=(grid,),
        in_specs=[
            pl.BlockSpec((_BLK, 3), lambda i: (i, 0)),
            _full((1, 3)),
            _full((3, D)), _full((1, D)),
            _full((D, D)), _full((1, D)),
            _full((3, D)), _full((1, D)),
            _full((D, D)), _full((1, D)),
        ],
        out_specs=pl.BlockSpec((_BLK, D), lambda i: (i, 0)),
        out_shape=jax.ShapeDtypeStruct((NP, D), _f32),
    )(xp, gf.reshape(1, 3),
      p['Wn1'], p['bn1'].reshape(1, D), p['Wn2'], p['bn2'].reshape(1, D),
      p['Wg1'], p['bg1'].reshape(1, D), p['Wg2'], p['bg2'].reshape(1, D))


def _qkv_body(h, wq, bq, wk, bk, wv, bv, ws, bs, qo, kvo, so):
    hh = h[...]
    q = jnp.dot(hh, wq[...], preferred_element_type=_f32) + bq[...]
    k = jnp.dot(hh, wk[...], preferred_element_type=_f32) + bk[...]
    v = jnp.dot(hh, wv[...], preferred_element_type=_f32) + bv[...]
    qo[...] = jnp.concatenate([q, jnp.zeros_like(q)], axis=1)
    kvo[...] = jnp.concatenate([k, v], axis=1)
    so[...] = jnp.dot(hh, ws[...], preferred_element_type=_f32) + bs[...]


def _qkv(h, wq, bq, wk, bk, wv, bv, ws, bs):
    grid = NP // _BLK
    spec = pl.BlockSpec((_BLK, D), lambda i: (i, 0))
    spec2 = pl.BlockSpec((_BLK, 2 * D), lambda i: (i, 0))
    return pl.pallas_call(
        _qkv_body,
        grid=(grid,),
        in_specs=[spec] + [_full((D, D)), _full((1, D))] * 4,
        out_specs=[spec2, spec2, spec],
        out_shape=[jax.ShapeDtypeStruct((NP, 2 * D), _f32),
                   jax.ShapeDtypeStruct((NP, 2 * D), _f32),
                   jax.ShapeDtypeStruct((NP, D), _f32)],
    )(h, wq, bq.reshape(1, D), wk, bk.reshape(1, D),
      wv, bv.reshape(1, D), ws, bs.reshape(1, D))


_BLKE = 1024


def _edge_body(qg, kvg, ef, we, sel, selt, msg_o, w_o):
    ee = jnp.dot(ef[...], we[...], preferred_element_type=_f32)
    q = qg[...][:, :D]
    k = kvg[...][:, :D]
    v = kvg[...][:, D:]
    kj = k + ee
    p = q * kj
    s = jnp.dot(p, sel[...], preferred_element_type=_f32) * (1.0 / jnp.sqrt(float(C)))
    w = jnp.exp(s)
    wb = jnp.dot(w, selt[...], preferred_element_type=_f32)
    msg_o[...] = (v + ee) * wb
    w_o[...] = jnp.concatenate([w, jnp.zeros((w.shape[0], 16 - H), _f32)], axis=1)


def _edge_math(qg, kvg, efp, we, sel, selt):
    grid = EP // _BLKE
    spec64 = pl.BlockSpec((_BLKE, D), lambda i: (i, 0))
    spec128 = pl.BlockSpec((_BLKE, 2 * D), lambda i: (i, 0))
    return pl.pallas_call(
        _edge_body,
        grid=(grid,),
        in_specs=[spec128, spec128,
                  pl.BlockSpec((_BLKE, 2), lambda i: (i, 0)),
                  _full((2, D)), _full((D, H)), _full((H, D))],
        out_specs=[spec64, pl.BlockSpec((_BLKE, 16), lambda i: (i, 0))],
        out_shape=[jax.ShapeDtypeStruct((EP, D), _f32),
                   jax.ShapeDtypeStruct((EP, 16), _f32)],
    )(qg, kvg, efp, we, sel, selt)


def _cumsum_body(msg, w, mu, out, carry):
    i = pl.program_id(0)

    @pl.when(i == 0)
    def _zero():
        out[...] = jnp.zeros_like(out)
        carry[...] = jnp.zeros_like(carry)

    @pl.when(i > 0)
    def _accum():
        x = jnp.concatenate(
            [msg[...], w[...], jnp.zeros((_BLKE, 128 - D - 16), _f32)], axis=1)
        x = x - mu[...]
        k = 1
        while k < _BLKE:
            shifted = jnp.concatenate(
                [jnp.zeros((k, 128), _f32), x[:_BLKE - k, :]], axis=0)
            x = x + shifted
            k *= 2
        x = x + carry[...]
        out[...] = x
        carry[...] = x[_BLKE - 1:_BLKE, :]


def _cumsum(msg, w, mu):
    grid = CROWS // _BLKE
    return pl.pallas_call(
        _cumsum_body,
        grid=(grid,),
        in_specs=[
            pl.BlockSpec((_BLKE, D), lambda i: (jnp.maximum(i - 1, 0), 0)),
            pl.BlockSpec((_BLKE, 16), lambda i: (jnp.maximum(i - 1, 0), 0)),
            _full((1, 128)),
        ],
        out_specs=pl.BlockSpec((_BLKE, 128), lambda i: (i, 0)),
        out_shape=jax.ShapeDtypeStruct((CROWS, 128), _f32),
        scratch_shapes=[pltpu.VMEM((1, 128), _f32)],
    )(msg, w, mu)


def _combine_body(chi, clo, mu, deg, selt16, skip, out):
    seg = chi[...] - clo[...] + deg[...] * mu[...]
    numer = seg[:, :D]
    den = jnp.dot(seg[:, D:D + 16], selt16[...], preferred_element_type=_f32)
    out[...] = jnp.maximum(numer / (den + 1e-16) + skip[...], 0.0)


def _combine(chi, clo, mu, deg, selt16, skip):
    grid = NP // _BLK
    spec128 = pl.BlockSpec((_BLK, 128), lambda i: (i, 0))
    spec = pl.BlockSpec((_BLK, D), lambda i: (i, 0))
    return pl.pallas_call(
        _combine_body,
        grid=(grid,),
        in_specs=[spec128, spec128, _full((1, 128)),
                  pl.BlockSpec((_BLK, 1), lambda i: (i, 0)),
                  _full((16, D)), spec],
        out_specs=spec,
        out_shape=jax.ShapeDtypeStruct((NP, D), _f32),
    )(chi, clo, mu, deg, selt16, skip)


def _tail_body(h, wir, bir, wiz, biz, win, bin_, bhr, bhz, bhn,
               wd1, bd1, wd2, bd2, wd3, bd3, o_o, mem_o):
    hh = h[...]
    r = jax.nn.sigmoid(jnp.dot(hh, wir[...], preferred_element_type=_f32)
                       + bir[...] + bhr[...])
    z = jax.nn.sigmoid(jnp.dot(hh, wiz[...], preferred_element_type=_f32)
                       + biz[...] + bhz[...])
    nn_ = jnp.tanh(jnp.dot(hh, win[...], preferred_element_type=_f32)
                   + bin_[...] + r * bhn[...])
    mem = (1.0 - z) * nn_
    mem_o[...] = mem
    o = jnp.maximum(jnp.dot(mem, wd1[...], preferred_element_type=_f32)
                    + bd1[...], 0.0)
    o = jnp.maximum(jnp.dot(o, wd2[...], preferred_element_type=_f32)
                    + bd2[...], 0.0)
    o_o[...] = jnp.dot(o, wd3[...], preferred_element_type=_f32) + bd3[...]


def _tail(h, p):
    wih = p['Wih']
    bih = p['bih']
    bhh = p['bhh']
    grid = NP // _BLK
    spec = pl.BlockSpec((_BLK, D), lambda i: (i, 0))
    wd3 = jnp.zeros((D, 8), _f32).at[:, :3].set(p['Wd3'])
    bd3 = jnp.zeros((1, 8), _f32).at[0, :3].set(p['bd3'])
    return pl.pallas_call(
        _tail_body,
        grid=(grid,),
        in_specs=[spec,
                  _full((D, D)), _full((1, D)),
                  _full((D, D)), _full((1, D)),
                  _full((D, D)), _full((1, D)),
                  _full((1, D)), _full((1, D)), _full((1, D)),
                  _full((D, D)), _full((1, D)),
                  _full((D, D)), _full((1, D)),
                  _full((D, 8)), _full((1, 8))],
        out_specs=[pl.BlockSpec((_BLK, 8), lambda i: (i, 0)), spec],
        out_shape=[jax.ShapeDtypeStruct((NP, 8), _f32),
                   jax.ShapeDtypeStruct((NP, D), _f32)],
    )(h,
      wih[:, 0:D], bih[0:D].reshape(1, D),
      wih[:, D:2 * D], bih[D:2 * D].reshape(1, D),
      wih[:, 2 * D:], bih[2 * D:].reshape(1, D),
      bhh[0:D].reshape(1, D), bhh[D:2 * D].reshape(1, D),
      bhh[2 * D:].reshape(1, D),
      p['Wd1'], p['bd1'].reshape(1, D), p['Wd2'], p['bd2'].reshape(1, D),
      wd3, bd3)


def _sc_mesh():
    return plsc.VectorSubcoreMesh(core_axis_name="c", subcore_axis_name="s",
                                  num_cores=2, num_subcores=16)


def _make_gather(n_groups, kb, rows_a, rows_b):
    tile_groups = n_groups // 32
    iters = tile_groups // kb

    def body(a_hbm, b_hbm, ia_hbm, ib_hbm, oa_hbm, ob_hbm,
             abuf, bbuf, ab, bb, sem):
        wid = lax.axis_index("s") * 2 + lax.axis_index("c")
        gbase = wid * tile_groups

        def step(t, carry):
            g0 = gbase + t * kb
            pltpu.sync_copy(ia_hbm.at[pl.ds(g0, kb)], abuf)
            pltpu.sync_copy(ib_hbm.at[pl.ds(g0, kb)], bbuf)
            handles = []
            for j in range(kb):
                handles.append(pltpu.async_copy(a_hbm.at[abuf.at[j]], ab.at[j], sem))
                handles.append(pltpu.async_copy(b_hbm.at[bbuf.at[j]], bb.at[j], sem))
            for hh in handles:
                hh.wait()
            pltpu.sync_copy(ab, oa_hbm.at[pl.ds(g0, kb)])
            pltpu.sync_copy(bb, ob_hbm.at[pl.ds(g0, kb)])
            return carry

        lax.fori_loop(0, iters, step, 0)

    return pl.kernel(
        body,
        out_type=[jax.ShapeDtypeStruct((n_groups, 128, 128), _f32)] * 2,
        mesh=_sc_mesh(),
        scratch_types=[
            pltpu.VMEM((kb, 128), _i32),
            pltpu.VMEM((kb, 128), _i32),
            pltpu.VMEM((kb, 128, 128), _f32),
            pltpu.VMEM((kb, 128, 128), _f32),
            pltpu.SemaphoreType.DMA,
        ],
    )


def _sc_gather_qkv(q, kv, dst2d, src2d):
    fn = _make_gather(NG, 2, NP, NP)
    return fn(q, kv, dst2d, src2d)


def _sc_gather_prefix(cg, ihi2d, ilo2d):
    fn = _make_gather(NGC, 1, CROWS, CROWS)
    return fn(cg, cg, ihi2d, ilo2d)


def kernel(x, edge_index, edge_features, global_features, params):
    p = params
    src = edge_index[0]
    dst = edge_index[1]
    srcp = jnp.concatenate([src, jnp.zeros((EP - E,), _i32)])
    dstp = jnp.concatenate([dst, jnp.full((EP - E,), N, _i32)])
    efp = jnp.concatenate([edge_features, jnp.zeros((EP - E, 2), _f32)])
    perm = jnp.argsort(dstp)
    dsts = dstp[perm]
    srcs = srcp[perm]
    efs = efp[perm]
    src2d = srcs.reshape(NG, 128)
    dst2d = dsts.reshape(NG, 128)
    bounds = jnp.searchsorted(dsts, jnp.arange(NP + 1, dtype=_i32)).astype(_i32)
    e_lo = bounds[:NP]
    e_hi = bounds[1:]
    deg = (e_hi - e_lo).astype(_f32).reshape(NP, 1)
    pad_idx = jnp.zeros((NGC * 128 - NP,), _i32)
    ihi2d = jnp.concatenate([e_hi + (_BLKE - 1), pad_idx]).reshape(NGC, 128)
    ilo2d = jnp.concatenate([e_lo + (_BLKE - 1), pad_idx]).reshape(NGC, 128)

    xp = jnp.zeros((NP, 3), _f32).at[:N].set(x)
    sel = jnp.asarray(np.repeat(np.eye(H, dtype=np.float32), C, axis=0))
    selt = jnp.asarray(sel.T)
    selt16 = jnp.zeros((16, D), _f32).at[:H].set(selt)
    mu = jnp.concatenate([jnp.zeros((1, D), _f32), jnp.ones((1, H), _f32),
                          jnp.zeros((1, 128 - D - H), _f32)], axis=1)

    h = _encoder(xp, global_features, p)
    for i in range(NUM_CONVS):
        q, kv, skip = _qkv(h, p['Wq'][i], p['bq'][i], p['Wk'][i], p['bk'][i],
                           p['Wv'][i], p['bv'][i], p['Wskip'][i], p['bskip'][i])
        qg, kvg = _sc_gather_qkv(q, kv, dst2d, src2d)
        msg, w = _edge_math(qg.reshape(EP, 2 * D), kvg.reshape(EP, 2 * D),
                            efs, p['We'][i], sel, selt)
        cg = _cumsum(msg, w, mu)
        chi, clo = _sc_gather_prefix(cg, ihi2d, ilo2d)
        h = _combine(chi.reshape(NGC * 128, 128),
                     clo.reshape(NGC * 128, 128),
                     mu, deg, selt16, skip)
    o, mem = _tail(h, p)
    return o[:N, :3], mem[:N]
